# baseline (speedup 1.0000x reference)
import functools
from math import sqrt

import jax
import jax.numpy as jnp
from jax import lax
from jax.experimental import pallas as pl
from jax.experimental.pallas import tpu as pltpu
from jax.experimental.pallas import tpu_sc as plsc

VOCAB_SIZE = 1000000
EMBEDDING_DIM = 64
SCALE = sqrt(EMBEDDING_DIM)

_INFO = plsc.get_sparse_core_info()
_NC = _INFO.num_cores
_NS = _INFO.num_subcores
_NW = _NC * _NS
_L = _INFO.num_lanes

_NBLK = VOCAB_SIZE // 128
_TAIL0 = _NBLK * 128
_BPW = -(-_NBLK // _NW)

_NI = 4096
_NJ = 200
_NIB = _NI // 128
_NSLAB = _NJ * _NIB
_SPW = _NSLAB // _NW


def _reformat_body(tT, tail2, t2, slab, buf2, lanes,
                   l0, l1, s0, s1):
    lsem = (l0, l1)
    ssem = (s0, s1)
    wid = lax.axis_index("s") * _NC + lax.axis_index("c")


    @pl.when(wid == 0)
    def _():
        pltpu.sync_copy(tail2, buf2.at[0, pl.ds(0, 32)])
        pltpu.sync_copy(buf2.at[0, pl.ds(0, 32)], t2.at[pl.ds(_TAIL0 // 2, 32)])

    def start_load(rb, b):
        @pl.when(rb < _NBLK)
        def _():
            pltpu.async_copy(tT.at[:, pl.ds(rb * 128, 128)],
                             slab.at[b, :, pl.ds(0, 128)], lsem[b])

    def wait_load(b):
        pltpu.make_async_copy(tT.at[:, pl.ds(0, 128)],
                              slab.at[b, :, pl.ds(0, 128)], lsem[b]).wait()

    def start_store(rb, b):
        pltpu.async_copy(buf2.at[b], t2.at[pl.ds(rb * 64, 64)], ssem[b])

    def wait_store(b):
        pltpu.make_async_copy(buf2.at[b], t2.at[pl.ds(0, 64)], ssem[b]).wait()

    start_load(wid, 0)

    def step(i2, carry):
        for b in range(2):
            i = i2 * 2 + b
            rb = wid + i * _NW

            @pl.when(rb < _NBLK)
            def _():
                start_load(rb + _NW, 1 - b)
                wait_load(b)

                @pl.when(i >= 2)
                def _():
                    wait_store(b)

                iota = lax.iota(jnp.int32, _L)
                zero = iota * 0
                rowbs = [iota + zb * _L for zb in range(4)]

                @plsc.parallel_loop(0, 64, unroll=2)
                def _(q):
                    c0 = zero + 2 * q
                    c1 = c0 + 1
                    vs = [plsc.load_gather(slab.at[b], [rowbs[zb], csp])
                          for csp in (c0, c1) for zb in range(4)]
                    for h in range(2):
                        for zb in range(4):
                            buf2[b, q, pl.ds(h * 64 + zb * _L, _L)] = (
                                vs[h * 4 + zb] * SCALE)

                start_store(rb, b)
        return carry

    lax.fori_loop(0, (_BPW + 1) // 2, step, 0)

    for b in range(2):
        last = wid + (_BPW - 2 + b) * _NW

        @pl.when(last < _NBLK)
        def _():
            wait_store(b)


def _gather_body(t2, xT, out, idxb, idx2, colb, rows, slab2,
                 g0, g1, o0, o1):
    gsem = (g0, g1)
    osem = (o0, o1)
    wid = lax.axis_index("s") * _NC + lax.axis_index("c")
    iota = lax.iota(jnp.int32, _L)

    def coords(s):
        sid = wid * _SPW + s
        return sid // _NIB, (sid % _NIB) * 128

    def prep_and_gather(s, b):
        j, i0 = coords(s)
        pltpu.sync_copy(xT.at[j, pl.ds(i0, 128)], idxb.at[b])
        for k in range(8):
            iv = idxb[b, pl.ds(k * _L, _L)]
            idx2[b, pl.ds(k * _L, _L)] = iv >> 1
            colb[b, pl.ds(k * _L, _L)] = (iv & 1) * 64
        pltpu.async_copy(t2.at[idx2.at[b]], rows.at[b, :, pl.ds(0, 128)],
                         gsem[b])

    def wait_gather(b):
        pltpu.make_async_copy(t2.at[idx2.at[b]], rows.at[b, :, pl.ds(0, 128)],
                              gsem[b]).wait()

    def start_out(s, b):
        j, i0 = coords(s)
        pltpu.async_copy(slab2.at[b],
                         out.at[j, :, pl.ds(i0, 128)], osem[b])

    def wait_out(b):
        pltpu.make_async_copy(slab2.at[b],
                              out.at[0, :, pl.ds(0, 128)], osem[b]).wait()

    prep_and_gather(0, 0)

    def step(s2, carry):
        for b in range(2):
            s = s2 * 2 + b

            @pl.when(s + 1 < _SPW)
            def _():
                prep_and_gather(s + 1, 1 - b)

            wait_gather(b)

            @pl.when(s >= 2)
            def _():
                wait_out(b)

            rowks = [iota + k * _L for k in range(8)]
            cks = [colb[b, pl.ds(k * _L, _L)] for k in range(8)]

            @plsc.parallel_loop(0, EMBEDDING_DIM, unroll=2)
            def _(c):
                vs = [plsc.load_gather(rows.at[b], [rowks[k], cks[k] + c])
                      for k in range(8)]
                for k in range(8):
                    slab2[b, c, pl.ds(k * _L, _L)] = vs[k]

            start_out(s, b)
        return carry

    lax.fori_loop(0, _SPW // 2, step, 0)
    for b in range(2):
        wait_out(b)


@jax.jit
def _launch(x, table):
    mesh = plsc.VectorSubcoreMesh(core_axis_name="c", subcore_axis_name="s")
    tT = jnp.swapaxes(table, 0, 1)
    xT = jnp.swapaxes(x, 0, 1).astype(jnp.int32)
    tail2 = (table[_TAIL0:] * SCALE).reshape(32, 128)

    reformat = pl.kernel(
        _reformat_body,
        mesh=mesh,
        out_type=jax.ShapeDtypeStruct((VOCAB_SIZE // 2, 128), jnp.float32),
        scratch_types=[
            pltpu.VMEM((2, EMBEDDING_DIM, 129), jnp.float32),
            pltpu.VMEM((2, 64, 128), jnp.float32),
            pltpu.VMEM((2 * _L,), jnp.int32),
        ] + [pltpu.SemaphoreType.DMA] * 4,
        compiler_params=pltpu.CompilerParams(needs_layout_passes=False),
    )
    t2 = reformat(tT, tail2)

    gather = pl.kernel(
        _gather_body,
        mesh=mesh,
        out_type=jax.ShapeDtypeStruct((_NJ, EMBEDDING_DIM, _NI), jnp.float32),
        scratch_types=[
            pltpu.VMEM((2, 128), jnp.int32),
            pltpu.VMEM((2, 128), jnp.int32),
            pltpu.VMEM((2, 128), jnp.int32),
            pltpu.VMEM((2, 128, 129), jnp.float32),
            pltpu.VMEM((2, EMBEDDING_DIM, 128), jnp.float32),
        ] + [pltpu.SemaphoreType.DMA] * 4,
        compiler_params=pltpu.CompilerParams(needs_layout_passes=False),
    )
    outP = gather(t2, xT)
    return jnp.transpose(outP, (2, 0, 1))


def kernel(x, table):
    return _launch(x, table)


# speedup vs baseline: 1.6795x; 1.6795x over previous
import functools
from math import sqrt

import jax
import jax.numpy as jnp
from jax import lax
from jax.experimental import pallas as pl
from jax.experimental.pallas import tpu as pltpu
from jax.experimental.pallas import tpu_sc as plsc

VOCAB_SIZE = 1000000
EMBEDDING_DIM = 64
SCALE = sqrt(EMBEDDING_DIM)

_INFO = plsc.get_sparse_core_info()
_NC = _INFO.num_cores
_NS = _INFO.num_subcores
_NW = _NC * _NS
_L = _INFO.num_lanes

_B = 4096 * 200
_PER_W = _B // _NW
_C = 400
_NCHUNK = _PER_W // _C
_NBUF = 4
_LOOKAHEAD = 2


def _emb_body(table_hbm, idx_hbm, out_hbm, idx_all, rows,
              g0, g1, g2, g3, o0, o1, o2, o3):
    gsems = (g0, g1, g2, g3)
    osems = (o0, o1, o2, o3)
    wid = lax.axis_index("s") * _NC + lax.axis_index("c")
    base = wid * _PER_W

    pltpu.sync_copy(idx_hbm.at[pl.ds(base, _PER_W)], idx_all)

    def start_gather(g, b):
        pltpu.async_copy(
            table_hbm.at[idx_all.at[pl.ds(g * _C, _C)]], rows.at[b],
            gsems[b])

    def wait_gather(b):
        pltpu.make_async_copy(
            table_hbm.at[idx_all.at[pl.ds(0, _C)]], rows.at[b],
            gsems[b]).wait()

    def start_out(g, b):
        pltpu.async_copy(
            rows.at[b], out_hbm.at[pl.ds(base + g * _C, _C)], osems[b])

    def wait_out(b):
        pltpu.make_async_copy(
            rows.at[b], out_hbm.at[pl.ds(base, _C)], osems[b]).wait()

    for g in range(_LOOKAHEAD):
        start_gather(g, g % _NBUF)

    def step(i0, carry):
        for b in range(_NBUF):
            g = i0 * _NBUF + b
            bp = (b + _LOOKAHEAD) % _NBUF

            @pl.when(g + _LOOKAHEAD < _NCHUNK)
            def _():
                @pl.when(g >= _LOOKAHEAD)
                def _():
                    wait_out(bp)
                start_gather(g + _LOOKAHEAD, bp)

            wait_gather(b)

            @plsc.parallel_loop(0, _C, unroll=8)
            def _(j):
                for k in range(EMBEDDING_DIM // _L):
                    sl = pl.ds(k * _L, _L)
                    rows[b, j, sl] = rows[b, j, sl] * SCALE

            start_out(g, b)
        return carry

    lax.fori_loop(0, _NCHUNK // _NBUF, step, 0)

    for b in range(_NBUF):
        wait_out(b)


@jax.jit
def _launch(idx, table):
    mesh = plsc.VectorSubcoreMesh(core_axis_name="c", subcore_axis_name="s")
    f = pl.kernel(
        _emb_body,
        mesh=mesh,
        out_type=jax.ShapeDtypeStruct((_B, EMBEDDING_DIM), jnp.float32),
        scratch_types=[
            pltpu.VMEM((_PER_W,), jnp.int32),
            pltpu.VMEM((_NBUF, _C, EMBEDDING_DIM), jnp.float32),
        ] + [pltpu.SemaphoreType.DMA] * (2 * _NBUF),
        compiler_params=pltpu.CompilerParams(
            use_tc_tiling_on_sc=False,
            skip_device_barrier=True,
        ),
    )
    return f(table, idx)


def kernel(x, table):
    idx = x.reshape(-1).astype(jnp.int32)
    out = _launch(idx, table)
    return out.reshape(x.shape + (EMBEDDING_DIM,))
